# Initial kernel scaffold; baseline (speedup 1.0000x reference)
import jax
import jax.numpy as jnp
from jax.experimental import pallas as pl


def kernel(example, label, shifts, noise1, noise2, mus_label, mus_class, positional_embedding):
    raise NotImplementedError("write your pallas kernel here")


# speedup vs baseline: 10.7598x; 10.7598x over previous
import jax
import jax.numpy as jnp
import numpy as np
from jax import lax
from jax.experimental import pallas as pl
from jax.experimental.pallas import tpu as pltpu
from jax.experimental.pallas import tpu_sc as plsc

S = 1024
N = 50
NMAX = 64
D = 128
K = 1024
EPS = 0.1
E_FAC = float(1.0 / np.sqrt(1.0 + EPS ** 2))
C_NOISE = float(E_FAC * EPS / np.sqrt(D))
T = 3 * N + 2
W = 2 * NMAX + D
NC, NS = 2, 16
NW = NC * NS
SPW = S // NW
TPAD = 160
LANES = 16


def _sc_body(table, gidx, n1, n2, pe, out,
             idx_v, rows_v, n1_v, n2_v, pe_v, stage_v,
             sem_g, sem_n, sem_o):
    wid = lax.axis_index("s") * NC + lax.axis_index("c")
    base = wid * SPW

    pltpu.sync_copy(gidx.at[pl.ds(base * TPAD, SPW * TPAD)], idx_v)

    @pl.when(wid == 0)
    def _():
        pltpu.sync_copy(pe, pe_v)

    def zero_rows(t, carry):
        for k in range(W // LANES):
            stage_v[t, pl.ds(LANES * k, LANES)] = jnp.zeros((LANES,), jnp.float32)
        return carry

    lax.fori_loop(0, T, zero_rows, 0)

    def sample_step(s, carry):
        gs = base + s
        g1 = pltpu.make_async_copy(
            table.at[idx_v.at[pl.ds(s * TPAD, 80)]],
            rows_v.at[pl.ds(0, 80)], sem_g)
        g2 = pltpu.make_async_copy(
            table.at[idx_v.at[pl.ds(s * TPAD + 80, 80)]],
            rows_v.at[pl.ds(80, 80)], sem_g)
        g1.start()
        g2.start()
        c1 = pltpu.make_async_copy(n1.at[gs], n1_v, sem_n)
        c2 = pltpu.make_async_copy(n2.at[gs], n2_v, sem_n)
        c1.start()
        c2.start()

        @pl.when(s > 0)
        def _():
            pltpu.make_async_copy(stage_v, out.at[gs - 1], sem_o).wait()

        @pl.when((wid == 0) & (s == 1))
        def _():
            def rezero(t, c):
                for k in range(D // LANES):
                    stage_v[t, pl.ds(LANES * k, LANES)] = (
                        jnp.zeros((LANES,), jnp.float32))
                return c
            lax.fori_loop(0, T, rezero, 0)

        g1.wait()
        g2.wait()
        c1.wait()
        c2.wait()

        def triple(i, c):
            t0 = 3 * i
            for k in range(D // LANES):
                sl = pl.ds(LANES * k, LANES)
                so = pl.ds(D + LANES * k, LANES)
                stage_v[t0, so] = E_FAC * rows_v[t0, sl] + C_NOISE * n1_v[i, sl]
                stage_v[t0 + 1, so] = (
                    E_FAC * rows_v[t0 + 1, sl] + C_NOISE * n2_v[i, sl])
                stage_v[t0 + 2, so] = rows_v[t0 + 2, sl]
            return c

        lax.fori_loop(0, N, triple, 0)
        for k in range(D // LANES):
            sl = pl.ds(LANES * k, LANES)
            so = pl.ds(D + LANES * k, LANES)
            stage_v[3 * N, so] = E_FAC * rows_v[3 * N, sl] + C_NOISE * n1_v[N, sl]
            stage_v[3 * N + 1, so] = (
                E_FAC * rows_v[3 * N + 1, sl] + C_NOISE * n2_v[N, sl])

        @pl.when((wid == 0) & (s == 0))
        def _():
            def pe_row(t, c):
                for k in range(D // LANES):
                    sl = pl.ds(LANES * k, LANES)
                    stage_v[t, sl] = pe_v[t, sl]
                return c
            lax.fori_loop(0, T, pe_row, 0)

        pltpu.make_async_copy(stage_v, out.at[gs], sem_o).start()
        return carry

    lax.fori_loop(0, SPW, sample_step, 0)
    pltpu.make_async_copy(stage_v, out.at[base + SPW - 1], sem_o).wait()


_sc_call = pl.kernel(
    _sc_body,
    out_type=jax.ShapeDtypeStruct((S, T, W), jnp.float32),
    mesh=plsc.VectorSubcoreMesh(
        core_axis_name="c", subcore_axis_name="s",
        num_cores=NC, num_subcores=NS),
    scratch_types=[
        pltpu.VMEM((SPW * TPAD,), jnp.int32),
        pltpu.VMEM((TPAD, D), jnp.float32),
        pltpu.VMEM((N + 1, D), jnp.float32),
        pltpu.VMEM((N + 1, D), jnp.float32),
        pltpu.VMEM((T, D), jnp.float32),
        pltpu.VMEM((T, W), jnp.float32),
        pltpu.SemaphoreType.DMA,
        pltpu.SemaphoreType.DMA,
        pltpu.SemaphoreType.DMA,
    ],
)


def kernel(example, label, shifts, noise1, noise2, mus_label, mus_class,
           positional_embedding):
    example = example.astype(jnp.int32)
    label = label.astype(jnp.int32)
    e0 = example[:, 0::2]
    e1 = example[:, 1::2]
    lab = label[:, :N] + K
    trip = jnp.stack([e0[:, :N], e1[:, :N], lab], axis=2).reshape(S, 3 * N)
    gidx = jnp.concatenate(
        [trip, e0[:, N:], e1[:, N:], jnp.zeros((S, TPAD - T), jnp.int32)],
        axis=1).reshape(S * TPAD)
    table = jnp.concatenate([mus_class, mus_label], axis=0)
    pe_slice = lax.dynamic_slice(
        positional_embedding[0], (shifts[0], 0), (T, 2 * NMAX))
    return _sc_call(table, gidx, noise1, noise2, pe_slice)


# speedup vs baseline: 22.8887x; 2.1272x over previous
import jax
import jax.numpy as jnp
import numpy as np
from jax import lax
from jax.experimental import pallas as pl
from jax.experimental.pallas import tpu as pltpu
from jax.experimental.pallas import tpu_sc as plsc

S = 1024
N = 50
NMAX = 64
D = 128
K = 1024
EPS = 0.1
E_FAC = float(1.0 / np.sqrt(1.0 + EPS ** 2))
C_NOISE = float(E_FAC * EPS / np.sqrt(D))
T = 3 * N + 2
W = 2 * NMAX + D
NC, NS = 2, 16
NW = NC * NS
SPW = S // NW
TPAD = 160
LANES = 16


def _sc_body(table, gidx, n1, n2, pe, out,
             idx_v, rows_a, rows_b, n1_a, n1_b, n2_a, n2_b, zeros_v, pe_v,
             sg_a, sg_b, sn_a, sn_b, so_a, so_b):
    wid = lax.axis_index("s") * NC + lax.axis_index("c")
    base = wid * SPW

    pltpu.sync_copy(gidx.at[pl.ds(base * TPAD, SPW * TPAD)], idx_v)

    @pl.when(wid == 0)
    def _():
        pltpu.sync_copy(pe, pe_v)

    def zero_row(t, c):
        for k in range(D // LANES):
            zeros_v[t, pl.ds(LANES * k, LANES)] = jnp.zeros((LANES,), jnp.float32)
        return c

    lax.fori_loop(0, T, zero_row, 0)

    def in_copies(s, rows, nn1, nn2, sg, sn):
        return (
            pltpu.make_async_copy(
                table.at[idx_v.at[pl.ds(s * TPAD, 128)]],
                rows.at[pl.ds(0, 128)], sg),
            pltpu.make_async_copy(
                table.at[idx_v.at[pl.ds(s * TPAD + 128, T - 128)]],
                rows.at[pl.ds(128, T - 128)], sg),
            pltpu.make_async_copy(n1.at[base + s], nn1, sn),
            pltpu.make_async_copy(n2.at[base + s], nn2, sn),
        )

    def issue_in(s, rows, nn1, nn2, sg, sn):
        for c in in_copies(s, rows, nn1, nn2, sg, sn):
            c.start()

    def wait_in(s, rows, nn1, nn2, sg, sn):
        for c in in_copies(s, rows, nn1, nn2, sg, sn):
            c.wait()

    def add_noise(rows, nn1, nn2):
        def triple(i, c):
            t0 = 3 * i
            for k in range(D // LANES):
                sl = pl.ds(LANES * k, LANES)
                rows[t0, sl] = rows[t0, sl] + C_NOISE * nn1[i, sl]
                rows[t0 + 1, sl] = rows[t0 + 1, sl] + C_NOISE * nn2[i, sl]
            return c

        lax.fori_loop(0, N + 1, triple, 0)

    def out_copies(s, rows, so):
        return (
            pltpu.make_async_copy(
                rows, out.at[base + s, :, pl.ds(D, D)], so),
            pltpu.make_async_copy(
                zeros_v, out.at[base + s, :, pl.ds(0, D)], so),
        )

    def issue_out(s, rows, so):
        pltpu.make_async_copy(rows, out.at[base + s, :, pl.ds(D, D)], so).start()
        first = (wid == 0) & (s == 0)

        @pl.when(first)
        def _():
            pltpu.make_async_copy(pe_v, out.at[base + s, :, pl.ds(0, D)], so).start()

        @pl.when(jnp.logical_not(first))
        def _():
            pltpu.make_async_copy(zeros_v, out.at[base + s, :, pl.ds(0, D)], so).start()

    def wait_out(s, rows, so):
        for c in out_copies(s, rows, so):
            c.wait()

    bufs_a = (rows_a, n1_a, n2_a, sg_a, sn_a)
    bufs_b = (rows_b, n1_b, n2_b, sg_b, sn_b)

    issue_in(0, *bufs_a)

    def pair(j, c):
        s0 = 2 * j
        s1 = s0 + 1
        wait_in(s0, *bufs_a)
        add_noise(rows_a, n1_a, n2_a)
        issue_out(s0, rows_a, so_a)

        @pl.when(j > 0)
        def _():
            wait_out(s1 - 2, rows_b, so_b)

        issue_in(s1, *bufs_b)
        wait_in(s1, *bufs_b)
        add_noise(rows_b, n1_b, n2_b)
        issue_out(s1, rows_b, so_b)

        @pl.when(j < SPW // 2 - 1)
        def _():
            wait_out(s0, rows_a, so_a)
            issue_in(s0 + 2, *bufs_a)

        return c

    lax.fori_loop(0, SPW // 2, pair, 0)
    wait_out(SPW - 2, rows_a, so_a)
    wait_out(SPW - 1, rows_b, so_b)


_sc_call = pl.kernel(
    _sc_body,
    out_type=jax.ShapeDtypeStruct((S, T, W), jnp.float32),
    mesh=plsc.VectorSubcoreMesh(
        core_axis_name="c", subcore_axis_name="s",
        num_cores=NC, num_subcores=NS),
    scratch_types=[
        pltpu.VMEM((SPW * TPAD,), jnp.int32),
        pltpu.VMEM((T, D), jnp.float32),
        pltpu.VMEM((T, D), jnp.float32),
        pltpu.VMEM((N + 1, D), jnp.float32),
        pltpu.VMEM((N + 1, D), jnp.float32),
        pltpu.VMEM((N + 1, D), jnp.float32),
        pltpu.VMEM((N + 1, D), jnp.float32),
        pltpu.VMEM((T, D), jnp.float32),
        pltpu.VMEM((T, D), jnp.float32),
        pltpu.SemaphoreType.DMA,
        pltpu.SemaphoreType.DMA,
        pltpu.SemaphoreType.DMA,
        pltpu.SemaphoreType.DMA,
        pltpu.SemaphoreType.DMA,
        pltpu.SemaphoreType.DMA,
    ],
)


def kernel(example, label, shifts, noise1, noise2, mus_label, mus_class,
           positional_embedding):
    example = example.astype(jnp.int32)
    label = label.astype(jnp.int32)
    e0 = example[:, 0::2]
    e1 = example[:, 1::2]
    lab = label[:, :N] + K
    trip = jnp.stack([e0[:, :N], e1[:, :N], lab], axis=2).reshape(S, 3 * N)
    gidx = jnp.concatenate(
        [trip, e0[:, N:], e1[:, N:], jnp.zeros((S, TPAD - T), jnp.int32)],
        axis=1).reshape(S * TPAD)
    table = jnp.concatenate([E_FAC * mus_class, mus_label], axis=0)
    pe_slice = lax.dynamic_slice(
        positional_embedding[0], (shifts[0], 0), (T, 2 * NMAX))
    return _sc_call(table, gidx, noise1, noise2, pe_slice)
